# baseline (speedup 1.0000x reference)
import functools

import jax
import jax.numpy as jnp
from jax import lax
from jax.experimental import pallas as pl
from jax.experimental.pallas import tpu as pltpu
from jax.experimental.pallas import tpu_sc as plsc

VOCAB = 1000000
DIM = 64
B = 4096
L = 200

_INFO = plsc.get_sparse_core_info()
_NC = _INFO.num_cores
_NS = _INFO.num_subcores
_NW = _NC * _NS
_RW = B // _NW


@functools.partial(
    pl.kernel,
    mesh=plsc.VectorSubcoreMesh(core_axis_name="c", subcore_axis_name="s"),
    out_type=jax.ShapeDtypeStruct((B, L, 2 * DIM), jnp.float32),
    scratch_types=[
        pltpu.VMEM((_RW, L), jnp.int32),
        pltpu.VMEM((4, L, DIM), jnp.float32),
        [pltpu.SemaphoreType.DMA] * 4,
        [pltpu.SemaphoreType.DMA] * 4,
    ],
    compiler_params=pltpu.CompilerParams(use_tc_tiling_on_sc=False),
)
def _emb_lookup(words_hbm, table_hbm, out_hbm, idx_all, rows_v, sem_g, sem_w):
    wid = lax.axis_index("s") * _NC + lax.axis_index("c")
    base = wid * _RW

    def fire_gather(i, b):
        pltpu.async_copy(table_hbm.at[idx_all.at[i]], rows_v.at[b], sem_g[b])

    def fire_wb(i, b):
        pltpu.async_copy(
            rows_v.at[b], out_hbm.at[base + i, :, pl.ds(0, DIM)], sem_w[b])

    def drain_gather(b):
        pltpu.make_async_copy(
            out_hbm.at[0, :, pl.ds(0, DIM)], rows_v.at[b], sem_g[b]).wait()

    def drain_wb(b):
        pltpu.make_async_copy(
            rows_v.at[b], out_hbm.at[0, :, pl.ds(0, DIM)], sem_w[b]).wait()

    pltpu.sync_copy(words_hbm.at[pl.ds(base, _RW)], idx_all)

    fire_gather(0, 0)
    fire_gather(1, 1)
    fire_gather(2, 2)
    drain_gather(0)
    fire_wb(0, 0)
    fire_gather(3, 3)
    drain_gather(1)
    fire_wb(1, 1)

    def body(g, carry):
        for k in range(4):
            i = 4 * g + k
            drain_wb(k)
            fire_gather(i, k)
            kp = (k + 2) % 4
            drain_gather(kp)
            fire_wb(i - 2, kp)
        return carry

    lax.fori_loop(1, _RW // 4, body, 0)

    drain_gather(2)
    fire_wb(_RW - 2, 2)
    drain_gather(3)
    fire_wb(_RW - 1, 3)
    drain_wb(0)
    drain_wb(1)
    drain_wb(2)
    drain_wb(3)


def kernel(words, word_seq_lens, context_emb, chars, char_seq_lens, table):
    del word_seq_lens, context_emb, chars, char_seq_lens
    out = _emb_lookup(words.astype(jnp.int32), table)
    return out[:, :, :DIM]


# speedup vs baseline: 1.2762x; 1.2762x over previous
import functools

import jax
import jax.numpy as jnp
from jax import lax
from jax.experimental import pallas as pl
from jax.experimental.pallas import tpu as pltpu
from jax.experimental.pallas import tpu_sc as plsc

VOCAB = 1000000
DIM = 64
B = 4096
L = 200

_INFO = plsc.get_sparse_core_info()
_NC = _INFO.num_cores
_NS = _INFO.num_subcores
_NW = _NC * _NS
_RW = B // _NW


@functools.partial(
    pl.kernel,
    mesh=plsc.VectorSubcoreMesh(core_axis_name="c", subcore_axis_name="s"),
    out_type=jax.ShapeDtypeStruct((B, L, 2 * DIM), jnp.float32),
    scratch_types=[
        pltpu.VMEM((_RW, L), jnp.int32),
        pltpu.VMEM((4, L, 2 * DIM), jnp.float32),
        [pltpu.SemaphoreType.DMA] * 4,
        [pltpu.SemaphoreType.DMA] * 4,
    ],
    compiler_params=pltpu.CompilerParams(use_tc_tiling_on_sc=False),
)
def _emb_lookup(words_hbm, table_hbm, out_hbm, idx_all, rows_v, sem_g, sem_w):
    wid = lax.axis_index("s") * _NC + lax.axis_index("c")
    base = wid * _RW

    def fire_gather(i, b):
        pltpu.async_copy(table_hbm.at[idx_all.at[i]], rows_v.at[b], sem_g[b])

    def fire_wb(i, b):
        pltpu.async_copy(rows_v.at[b], out_hbm.at[base + i], sem_w[b])

    def drain_gather(b):
        pltpu.make_async_copy(out_hbm.at[0], rows_v.at[b], sem_g[b]).wait()

    def drain_wb(b):
        pltpu.make_async_copy(rows_v.at[b], out_hbm.at[0], sem_w[b]).wait()

    pltpu.sync_copy(words_hbm.at[pl.ds(base, _RW)], idx_all)

    fire_gather(0, 0)
    fire_gather(1, 1)
    fire_gather(2, 2)
    drain_gather(0)
    fire_wb(0, 0)
    fire_gather(3, 3)
    drain_gather(1)
    fire_wb(1, 1)

    def body(g, carry):
        for k in range(4):
            i = 4 * g + k
            drain_wb(k)
            fire_gather(i, k)
            kp = (k + 2) % 4
            drain_gather(kp)
            fire_wb(i - 2, kp)
        return carry

    lax.fori_loop(1, _RW // 4, body, 0)

    drain_gather(2)
    fire_wb(_RW - 2, 2)
    drain_gather(3)
    fire_wb(_RW - 1, 3)
    drain_wb(0)
    drain_wb(1)
    drain_wb(2)
    drain_wb(3)


def kernel(words, word_seq_lens, context_emb, chars, char_seq_lens, table):
    del word_seq_lens, context_emb, chars, char_seq_lens
    tpad = table @ jnp.eye(DIM, 2 * DIM, dtype=table.dtype)
    out = _emb_lookup(words.astype(jnp.int32), tpad)
    return out[:, :, :DIM]
